# baseline (speedup 1.0000x reference)
import functools

import jax
import jax.numpy as jnp
from jax.experimental import pallas as pl

P, V, C = 50000, 5, 128
NQ = 4
BQ = 500
BP = NQ * BQ
F32 = jnp.float32


def _block(xq, wd1, wn1, b1, wd2, wn2, b2, wf1, bf1, g, beta, wf2, bf2):
    hv = [xq[:, v, :] for v in range(V)]

    agg1 = jnp.dot(sum(hv), wn1, preferred_element_type=F32) + b1
    h1 = [jnp.maximum(jnp.dot(h, wd1, preferred_element_type=F32) + agg1, 0.0)
          for h in hv]

    agg2 = jnp.dot(sum(h1), wn2, preferred_element_type=F32) + b2
    h2 = [jnp.maximum(jnp.dot(h, wd2, preferred_element_type=F32) + agg2, 0.0)
          for h in h1]

    cand = h2[0]
    for h in h2[1:]:
        cand = jnp.maximum(cand, h)

    z = jnp.dot(cand, wf1, preferred_element_type=F32) + bf1
    mu = jnp.mean(z, axis=-1, keepdims=True)
    var = jnp.mean((z - mu) * (z - mu), axis=-1, keepdims=True)
    z = (z - mu) * jax.lax.rsqrt(var + 1e-5) * g + beta
    z = jnp.maximum(z, 0.0)
    return jnp.dot(z, wf2, preferred_element_type=F32) + bf2


def _body(*refs):
    x_refs = refs[:NQ]
    (wd1_ref, wn1_ref, b1_ref, wd2_ref, wn2_ref, b2_ref,
     wf1_ref, bf1_ref, g_ref, beta_ref, wf2_ref, bf2_ref, out_ref) = refs[NQ:]
    ws = (wd1_ref[...], wn1_ref[...], b1_ref[...], wd2_ref[...], wn2_ref[...],
          b2_ref[...], wf1_ref[...], bf1_ref[...], g_ref[...], beta_ref[...],
          wf2_ref[...], bf2_ref[...])
    for q in range(NQ):
        out_ref[pl.ds(q * BQ, BQ), :] = _block(x_refs[q][...], *ws)


@functools.partial(jax.jit, static_argnames=("interpret",))
def _run(x, wd1, wn1, b1, wd2, wn2, b2, wf1, bf1, g, beta, wf2, bf2,
         interpret=False):
    full = lambda shape: pl.BlockSpec(shape, lambda i: (0, 0))
    x_spec = lambda q: pl.BlockSpec(
        (BQ, V, C), lambda i, q=q: (i * NQ + q, 0, 0))
    return pl.pallas_call(
        _body,
        grid=(P // BP,),
        in_specs=[x_spec(q) for q in range(NQ)] + [
            full((C, C)), full((C, C)), full((1, C)),
            full((C, C)), full((C, C)), full((1, C)),
            full((C, C)), full((1, C)), full((1, C)), full((1, C)),
            full((C, 1)), full((1, 1)),
        ],
        out_specs=pl.BlockSpec((BP, 1), lambda i: (i, 0)),
        out_shape=jax.ShapeDtypeStruct((P, 1), F32),
        interpret=interpret,
    )(*([x] * NQ), wd1, wn1, b1, wd2, wn2, b2, wf1, bf1, g, beta, wf2, bf2)


def kernel(x, W1_self, W1_nbr, b1, W2_self, W2_nbr, b2, Wf1, bf1,
           ln_gamma, ln_beta, Wf2, bf2):
    return _run(
        x,
        W1_self - W1_nbr, W1_nbr, b1.reshape(1, C),
        W2_self - W2_nbr, W2_nbr, b2.reshape(1, C),
        Wf1, bf1.reshape(1, C), ln_gamma.reshape(1, C), ln_beta.reshape(1, C),
        Wf2, bf2.reshape(1, 1),
    )


# speedup vs baseline: 1.0768x; 1.0768x over previous
import functools

import jax
import jax.numpy as jnp
from jax.experimental import pallas as pl

P, V, C = 50000, 5, 128
NQ = 1
BQ = 2000
BP = NQ * BQ
F32 = jnp.float32


def _block(xq, wd1, wn1, b1, wd2, wn2, b2, wf1, bf1, g, beta, wf2, bf2):
    hv = [xq[:, v, :] for v in range(V)]

    agg1 = jnp.dot(sum(hv), wn1, preferred_element_type=F32) + b1
    h1 = [jnp.maximum(jnp.dot(h, wd1, preferred_element_type=F32) + agg1, 0.0)
          for h in hv]

    agg2 = jnp.dot(sum(h1), wn2, preferred_element_type=F32) + b2
    h2 = [jnp.maximum(jnp.dot(h, wd2, preferred_element_type=F32) + agg2, 0.0)
          for h in h1]

    cand = h2[0]
    for h in h2[1:]:
        cand = jnp.maximum(cand, h)

    z = jnp.dot(cand, wf1, preferred_element_type=F32) + bf1
    mu = jnp.mean(z, axis=-1, keepdims=True)
    var = jnp.mean((z - mu) * (z - mu), axis=-1, keepdims=True)
    z = (z - mu) * jax.lax.rsqrt(var + 1e-5) * g + beta
    z = jnp.maximum(z, 0.0)
    return jnp.dot(z, wf2, preferred_element_type=F32) + bf2


def _body(*refs):
    x_refs = refs[:NQ]
    (wd1_ref, wn1_ref, b1_ref, wd2_ref, wn2_ref, b2_ref,
     wf1_ref, bf1_ref, g_ref, beta_ref, wf2_ref, bf2_ref, out_ref) = refs[NQ:]
    ws = (wd1_ref[...], wn1_ref[...], b1_ref[...], wd2_ref[...], wn2_ref[...],
          b2_ref[...], wf1_ref[...], bf1_ref[...], g_ref[...], beta_ref[...],
          wf2_ref[...], bf2_ref[...])
    for q in range(NQ):
        out_ref[pl.ds(q * BQ, BQ), :] = _block(x_refs[q][...], *ws)


@functools.partial(jax.jit, static_argnames=("interpret",))
def _run(x, wd1, wn1, b1, wd2, wn2, b2, wf1, bf1, g, beta, wf2, bf2,
         interpret=False):
    full = lambda shape: pl.BlockSpec(shape, lambda i: (0, 0))
    x_spec = lambda q: pl.BlockSpec(
        (BQ, V, C), lambda i, q=q: (i * NQ + q, 0, 0))
    return pl.pallas_call(
        _body,
        grid=(P // BP,),
        in_specs=[x_spec(q) for q in range(NQ)] + [
            full((C, C)), full((C, C)), full((1, C)),
            full((C, C)), full((C, C)), full((1, C)),
            full((C, C)), full((1, C)), full((1, C)), full((1, C)),
            full((C, 128)), full((1, 1)),
        ],
        out_specs=pl.BlockSpec((BP, 128), lambda i: (i, 0)),
        out_shape=jax.ShapeDtypeStruct((P, 128), F32),
        interpret=interpret,
    )(*([x] * NQ), wd1, wn1, b1, wd2, wn2, b2, wf1, bf1, g, beta, wf2, bf2)


def kernel(x, W1_self, W1_nbr, b1, W2_self, W2_nbr, b2, Wf1, bf1,
           ln_gamma, ln_beta, Wf2, bf2):
    scores = _run(
        x,
        W1_self - W1_nbr, W1_nbr, b1.reshape(1, C),
        W2_self - W2_nbr, W2_nbr, b2.reshape(1, C),
        Wf1, bf1.reshape(1, C), ln_gamma.reshape(1, C), ln_beta.reshape(1, C),
        jnp.tile(Wf2, (1, 128)), bf2.reshape(1, 1),
    )
    return scores[:, :1]
